# baseline (speedup 1.0000x reference)
import functools

import jax
import jax.numpy as jnp
from jax import lax
from jax.experimental import pallas as pl
from jax.experimental.pallas import tpu as pltpu
from jax.experimental.pallas import tpu_sc as plsc

NC = 2
NS = 16
NW = NC * NS

IDX_ROW = 512
CHUNK_ROWS = 1
CHUNK = IDX_ROW * CHUNK_ROWS
CBLK = CHUNK // 128


def _gather_body(table_hbm, idx_hbm, out_hbm,
                 idx_v, rows_v, tbuf, isem0, isem1, gsem0, gsem1,
                 osem0, osem1):
    D = table_hbm.shape[1]
    n_rows = idx_hbm.shape[0]
    rows_per_w = n_rows // NW
    chunks = rows_per_w // CHUNK_ROWS
    isems = (isem0, isem1)
    gsems = (gsem0, gsem1)
    osems = (osem0, osem1)

    wid = lax.axis_index("s") * NC + lax.axis_index("c")
    row0 = wid * rows_per_w
    iota16 = lax.iota(jnp.int32, 16)

    def idx_dma(c, b):
        pltpu.async_copy(
            idx_hbm.at[pl.ds(row0 + c * CHUNK_ROWS, CHUNK_ROWS)],
            idx_v.at[b], isems[b])

    def idx_start(c, b):
        @pl.when(c < chunks)
        def _():
            idx_dma(c, b)

    def idx_wait(b):
        pltpu.make_async_copy(
            idx_hbm.at[pl.ds(0, CHUNK_ROWS)], idx_v.at[b], isems[b]).wait()

    def fire_gathers(b):
        for j in range(CHUNK_ROWS):
            pltpu.async_copy(
                table_hbm.at[idx_v.at[b].at[j]],
                rows_v.at[b].at[pl.ds(j * IDX_ROW, IDX_ROW)], gsems[b])

    def wait_gathers(b):
        for j in range(CHUNK_ROWS):
            pltpu.make_async_copy(
                table_hbm.at[idx_v.at[b].at[j]],
                rows_v.at[b].at[pl.ds(j * IDX_ROW, IDX_ROW)], gsems[b]).wait()

    dcolv, octv, subv = [], [], []
    for c0 in (0, 16):
        for d0 in range(16):
            dc = lax.bitwise_and(iota16 + d0, 15) + c0
            dcolv.append(dc)
            octv.append(lax.shift_right_logical(dc, 3))
            subv.append(lax.bitwise_and(dc, 7))

    def transpose(b, tp):
        rows_buf = rows_v.at[b]
        tdst = tbuf.at[tp]

        def q_body(q, carry):
            cc = lax.shift_right_logical(q, 2)
            ccv = jnp.full((16,), 0, jnp.int32) + cc
            for h in range(2):
                lvec = (iota16 + 32 * lax.bitwise_and(q, 3) + 16 * h)
                ivec = lvec + cc * 128
                for t in range(2 * 16):
                    v = plsc.load_gather(rows_buf, [ivec, dcolv[t]])
                    plsc.store_scatter(
                        tdst, [octv[t], ccv, subv[t], lvec], v)
            return carry

        lax.fori_loop(0, CBLK * 4, q_body, 0)

    def out_pos(c):
        p0 = row0 * IDX_ROW + c * CHUNK
        jcol = lax.shift_right_logical(p0, 14)
        cblk = lax.shift_right_logical(
            lax.bitwise_and(p0, 16383), 7)
        return jcol, cblk

    def wb_start(c, tp):
        jcol, cblk = out_pos(c)
        pltpu.async_copy(
            tbuf.at[tp], out_hbm.at[jcol].at[:, pl.ds(cblk, CBLK)],
            osems[tp])

    def wb_wait(tp):
        pltpu.make_async_copy(
            tbuf.at[tp], out_hbm.at[0].at[:, pl.ds(0, CBLK)],
            osems[tp]).wait()

    idx_dma(0, 0)
    idx_dma(1, 1)
    idx_wait(0)
    fire_gathers(0)
    idx_wait(1)
    fire_gathers(1)
    wait_gathers(0)
    idx_start(2, 0)
    transpose(0, 0)
    wb_start(0, 0)

    def step(g, carry):
        for b in (0, 1):
            c = g * 2 + b
            idx_wait(b)
            fire_gathers(b)
            wait_gathers(1 - b)
            idx_start(c + 1, 1 - b)
            if b == 0:
                @pl.when(g >= 2)
                def _():
                    wb_wait(1)
            else:
                wb_wait(0)
            transpose(1 - b, 1 - b)
            wb_start(c - 1, 1 - b)
        return carry

    lax.fori_loop(1, chunks // 2, step, 0)

    wait_gathers(1)
    wb_wait(1)
    transpose(1, 1)
    wb_start(chunks - 1, 1)
    wb_wait(0)
    wb_wait(1)


@functools.partial(jax.jit, static_argnames=())
def kernel(input, embeddings):
    n_i, n_j = input.shape
    D = embeddings.shape[1]
    flat_idx = input.T.reshape(-1).astype(jnp.int32)
    B = flat_idx.shape[0]
    idx2d = flat_idx.reshape(B // IDX_ROW, IDX_ROW)

    mesh = plsc.VectorSubcoreMesh(
        core_axis_name="c", subcore_axis_name="s",
        num_cores=NC, num_subcores=NS,
    )
    out5 = pl.kernel(
        _gather_body,
        out_type=jax.ShapeDtypeStruct(
            (n_j, D // 8, n_i // 128, 8, 128), jnp.float32),
        mesh=mesh,
        scratch_types=[
            pltpu.VMEM((2, CHUNK_ROWS, IDX_ROW), jnp.int32),
            pltpu.VMEM((2, CHUNK, D), jnp.float32),
            pltpu.VMEM((2, D // 8, CBLK, 8, 128), jnp.float32),
            pltpu.SemaphoreType.DMA,
            pltpu.SemaphoreType.DMA,
            pltpu.SemaphoreType.DMA,
            pltpu.SemaphoreType.DMA,
            pltpu.SemaphoreType.DMA,
            pltpu.SemaphoreType.DMA,
        ],
        compiler_params=pltpu.CompilerParams(
            use_tc_tiling_on_sc=False, needs_layout_passes=False),
    )(embeddings, idx2d)
    return out5.transpose(2, 4, 0, 1, 3).reshape(n_i, n_j, D)


# speedup vs baseline: 1.1074x; 1.1074x over previous
import functools

import jax
import jax.numpy as jnp
from jax import lax
from jax.experimental import pallas as pl
from jax.experimental.pallas import tpu as pltpu
from jax.experimental.pallas import tpu_sc as plsc

NC = 2
NS = 16
NW = NC * NS

IDX_ROW = 512
CHUNK_ROWS = 1
CHUNK = IDX_ROW * CHUNK_ROWS
CBLK = CHUNK // 128


def _gather_body(table_hbm, idx_hbm, out_hbm,
                 idx_v, rows_v, tbuf, isem0, isem1, gsem0, gsem1,
                 osem0, osem1):
    D = table_hbm.shape[1]
    n_rows = idx_hbm.shape[0]
    rows_per_w = n_rows // NW
    chunks = rows_per_w // CHUNK_ROWS
    isems = (isem0, isem1)
    gsems = (gsem0, gsem1)
    osems = (osem0, osem1)

    wid = lax.axis_index("s") * NC + lax.axis_index("c")
    row0 = wid * rows_per_w
    iota16 = lax.iota(jnp.int32, 16)

    def idx_dma(c, b):
        pltpu.async_copy(
            idx_hbm.at[pl.ds(row0 + c * CHUNK_ROWS, CHUNK_ROWS)],
            idx_v.at[b], isems[b])

    def idx_start(c, b):
        @pl.when(c < chunks)
        def _():
            idx_dma(c, b)

    def idx_wait(b):
        pltpu.make_async_copy(
            idx_hbm.at[pl.ds(0, CHUNK_ROWS)], idx_v.at[b], isems[b]).wait()

    def fire_gathers(b):
        for j in range(CHUNK_ROWS):
            pltpu.async_copy(
                table_hbm.at[idx_v.at[b].at[j]],
                rows_v.at[b].at[pl.ds(j * IDX_ROW, IDX_ROW)], gsems[b])

    def wait_gathers(b):
        for j in range(CHUNK_ROWS):
            pltpu.make_async_copy(
                table_hbm.at[idx_v.at[b].at[j]],
                rows_v.at[b].at[pl.ds(j * IDX_ROW, IDX_ROW)], gsems[b]).wait()

    dcolv, octv, subv = [], [], []
    for c0 in (0, 16):
        for d0 in range(16):
            dc = lax.bitwise_and(iota16 + d0, 15) + c0
            dcolv.append(dc)
            octv.append(lax.shift_right_logical(dc, 3))
            subv.append(lax.bitwise_and(dc, 7))

    def transpose(b, tp):
        rows_buf = rows_v.at[b]
        tdst = tbuf.at[tp]

        def q_body(q, carry):
            cc = lax.shift_right_logical(q, 3)
            ccv = jnp.full((16,), 0, jnp.int32) + cc
            lvec = iota16 + 16 * lax.bitwise_and(q, 7)
            ivec = lvec + cc * 128
            for t in range(2 * 16):
                v = plsc.load_gather(rows_buf, [ivec, dcolv[t]])
                plsc.store_scatter(tdst, [octv[t], ccv, subv[t], lvec], v)
            return carry

        lax.fori_loop(0, CBLK * 8, q_body, 0)

    def out_pos(c):
        p0 = row0 * IDX_ROW + c * CHUNK
        jcol = lax.shift_right_logical(p0, 14)
        cblk = lax.shift_right_logical(
            lax.bitwise_and(p0, 16383), 7)
        return jcol, cblk

    def wb_start(c, tp):
        jcol, cblk = out_pos(c)
        pltpu.async_copy(
            tbuf.at[tp], out_hbm.at[jcol].at[:, pl.ds(cblk, CBLK)],
            osems[tp])

    def wb_wait(tp):
        pltpu.make_async_copy(
            tbuf.at[tp], out_hbm.at[0].at[:, pl.ds(0, CBLK)],
            osems[tp]).wait()

    idx_dma(0, 0)
    idx_dma(1, 1)
    idx_wait(0)
    fire_gathers(0)
    idx_wait(1)
    fire_gathers(1)
    wait_gathers(0)
    idx_start(2, 0)
    transpose(0, 0)
    wb_start(0, 0)

    def step(g, carry):
        for b in (0, 1):
            c = g * 2 + b
            idx_wait(b)
            fire_gathers(b)
            wait_gathers(1 - b)
            idx_start(c + 1, 1 - b)
            if b == 0:
                @pl.when(g >= 2)
                def _():
                    wb_wait(1)
            else:
                wb_wait(0)
            transpose(1 - b, 1 - b)
            wb_start(c - 1, 1 - b)
        return carry

    lax.fori_loop(1, chunks // 2, step, 0)

    wait_gathers(1)
    wb_wait(1)
    transpose(1, 1)
    wb_start(chunks - 1, 1)
    wb_wait(0)
    wb_wait(1)


@functools.partial(jax.jit, static_argnames=())
def kernel(input, embeddings):
    n_i, n_j = input.shape
    D = embeddings.shape[1]
    flat_idx = input.T.reshape(-1).astype(jnp.int32)
    B = flat_idx.shape[0]
    idx2d = flat_idx.reshape(B // IDX_ROW, IDX_ROW)

    mesh = plsc.VectorSubcoreMesh(
        core_axis_name="c", subcore_axis_name="s",
        num_cores=NC, num_subcores=NS,
    )
    out5 = pl.kernel(
        _gather_body,
        out_type=jax.ShapeDtypeStruct(
            (n_j, D // 8, n_i // 128, 8, 128), jnp.float32),
        mesh=mesh,
        scratch_types=[
            pltpu.VMEM((2, CHUNK_ROWS, IDX_ROW), jnp.int32),
            pltpu.VMEM((2, CHUNK, D), jnp.float32),
            pltpu.VMEM((2, D // 8, CBLK, 8, 128), jnp.float32),
            pltpu.SemaphoreType.DMA,
            pltpu.SemaphoreType.DMA,
            pltpu.SemaphoreType.DMA,
            pltpu.SemaphoreType.DMA,
            pltpu.SemaphoreType.DMA,
            pltpu.SemaphoreType.DMA,
        ],
        compiler_params=pltpu.CompilerParams(
            use_tc_tiling_on_sc=False, needs_layout_passes=False),
    )(embeddings, idx2d)
    return out5.transpose(2, 4, 0, 1, 3).reshape(n_i, n_j, D)
